# baseline (speedup 1.0000x reference)
import functools

import jax
import jax.numpy as jnp
from jax import lax
from jax.experimental import pallas as pl
from jax.experimental.pallas import tpu as pltpu
from jax.experimental.pallas import tpu_sc as plsc

_TOTAL = 4 * 4096 * 2048
_ROWS = 16384
_COLS = 2048
_NW = 32
_CHR = 8
_ROWS_W = _ROWS // _NW
_NCH = _ROWS_W // _CHR
_UNROLL = 8


def _sc_loss_partials(o2, t2):
    mesh = plsc.VectorSubcoreMesh(core_axis_name="c", subcore_axis_name="s")

    @functools.partial(
        pl.kernel,
        mesh=mesh,
        out_type=jax.ShapeDtypeStruct((_NW, 16), jnp.float32),
        scratch_types=[
            pltpu.VMEM((2, _CHR, _COLS), jnp.float32),
            pltpu.VMEM((2, _CHR, _COLS), jnp.float32),
            pltpu.VMEM((16,), jnp.float32),
            pltpu.SemaphoreType.DMA,
            pltpu.SemaphoreType.DMA,
            pltpu.SemaphoreType.DMA,
            pltpu.SemaphoreType.DMA,
        ],
    )
    def k(o_hbm, t_hbm, out_hbm, o_buf, t_buf, acc_vm, so0, so1, st0, st1):
        wid = lax.axis_index("s") * 2 + lax.axis_index("c")
        row0 = wid * _ROWS_W
        sems_o = (so0, so1)
        sems_t = (st0, st1)

        def copy_o(k_idx, b):
            return pltpu.make_async_copy(
                o_hbm.at[pl.ds(row0 + k_idx * _CHR, _CHR)], o_buf.at[b],
                sems_o[b])

        def copy_t(k_idx, b):
            return pltpu.make_async_copy(
                t_hbm.at[pl.ds(row0 + k_idx * _CHR, _CHR)], t_buf.at[b],
                sems_t[b])

        def start(k_idx, b):
            copy_o(k_idx, b).start()
            copy_t(k_idx, b).start()

        def wait(k_idx, b):
            copy_o(k_idx, b).wait()
            copy_t(k_idx, b).wait()

        def chunk_sum(b, accs):
            def vbody(v, a):
                out = list(a)
                for u in range(_UNROLL):
                    off = v * _UNROLL * 16 + u * 16
                    for r in range(_CHR):
                        o = o_buf.at[b].at[r][pl.ds(off, 16)]
                        t = t_buf.at[b].at[r][pl.ds(off, 16)]
                        d = jnp.where(t != 0.0, o - t, 0.0)
                        out[u] = out[u] + d * d
                return tuple(out)

            return lax.fori_loop(0, _COLS // (16 * _UNROLL), vbody, accs)

        start(0, 0)
        start(1, 1)

        def gbody(gg, accs):
            for b in (0, 1):
                k_idx = 2 * gg + b
                wait(k_idx, b)
                accs = chunk_sum(b, accs)
                start(k_idx + 2, b)
            return accs

        zero = jnp.zeros((16,), jnp.float32)
        accs = lax.fori_loop(0, (_NCH - 2) // 2, gbody, (zero,) * _UNROLL)
        for b in (0, 1):
            wait(_NCH - 2 + b, b)
            accs = chunk_sum(b, accs)

        acc = accs[0]
        for u in range(1, _UNROLL):
            acc = acc + accs[u]
        acc_vm[...] = acc
        pltpu.sync_copy(acc_vm, out_hbm.at[wid])

    return k(o2, t2)


def kernel(output, target):
    o2 = output.reshape(_ROWS, _COLS)
    t2 = target.reshape(_ROWS, _COLS)
    partials = _sc_loss_partials(o2, t2)
    return jnp.sum(partials) / _TOTAL
